# baseline (speedup 1.0000x reference)
import functools

import jax
import jax.numpy as jnp
from jax import lax
from jax.experimental import pallas as pl
from jax.experimental.pallas import tpu as pltpu
from jax.experimental.pallas import tpu_sc as plsc

_BB = 1024


def _sc_gather(table, idx):
    b, d = idx.shape[0], table.shape[1]
    info = plsc.get_sparse_core_info()
    ncores = info.num_cores
    nwork = ncores * info.num_subcores
    bpw = b // nwork
    mesh = plsc.VectorSubcoreMesh(core_axis_name="c", subcore_axis_name="s")

    @functools.partial(
        pl.kernel,
        mesh=mesh,
        out_type=jax.ShapeDtypeStruct((b, d), table.dtype),
        scratch_types=[
            pltpu.VMEM((bpw,), jnp.int32),
            pltpu.VMEM((bpw, d), table.dtype),
            pltpu.SemaphoreType.DMA,
        ],
    )
    def gather_kernel(table_hbm, idx_hbm, out_hbm, idx_v, rows_v, sem):
        wid = lax.axis_index("s") * ncores + lax.axis_index("c")
        base = wid * bpw
        pltpu.sync_copy(idx_hbm.at[pl.ds(base, bpw)], idx_v)
        pltpu.async_copy(table_hbm.at[idx_v], rows_v, sem).wait()
        pltpu.sync_copy(rows_v, out_hbm.at[pl.ds(base, bpw)])

    return gather_kernel(table, idx)


def _fold_body(wf, bf, wm, bm, ae, ge, wout, bo,
               wfa, wmm, wu, tage, tgen, btot):
    h = wf.shape[1]
    w = wout[...]
    wag = w[h:2 * h, :]
    wg = w[2 * h:3 * h, :]
    wa = w[3 * h:4 * h, :]
    wm2 = w[4 * h:5 * h, :]
    wfa[...] = jnp.dot(wf[...], wa,
                       preferred_element_type=jnp.float32).astype(jnp.bfloat16)
    wmm[...] = jnp.dot(wm[...], wm2,
                       preferred_element_type=jnp.float32).astype(jnp.bfloat16)
    wu[...] = w[0:h, :].astype(jnp.bfloat16)
    tage[...] = jnp.dot(ae[...], wag, preferred_element_type=jnp.float32)
    tgen[...] = jnp.dot(ge[...], wg, preferred_element_type=jnp.float32)
    btot[...] = (bo[...]
                 + jnp.dot(bf[...], wa, preferred_element_type=jnp.float32)
                 + jnp.dot(bm[...], wm2, preferred_element_type=jnp.float32))


def _main_body(art, mom, u, age, gen, wfa, wmm, wu, tage, tgen, btot, out):
    acc = jnp.dot(art[...].astype(jnp.bfloat16), wfa[...],
                  preferred_element_type=jnp.float32)
    acc = acc + jnp.dot(mom[...].astype(jnp.bfloat16), wmm[...],
                        preferred_element_type=jnp.float32)
    acc = acc + jnp.dot(u[...].astype(jnp.bfloat16), wu[...],
                        preferred_element_type=jnp.float32)
    ids = lax.broadcasted_iota(jnp.int32, (_BB, 8), 1)
    oh_age = (age[...] == ids).astype(jnp.float32)
    acc = acc + jnp.dot(oh_age, tage[...], preferred_element_type=jnp.float32)
    oh_gen = (gen[...] == ids).astype(jnp.float32)
    acc = acc + jnp.dot(oh_gen, tgen[...], preferred_element_type=jnp.float32)
    out[...] = acc + btot[...]


def kernel(articles, moments, uid, age, gender, W_feat, b_feat, W_mom, b_mom,
           user_emb, age_emb, gender_emb, W_out, b_out):
    b, a_dim = articles.shape
    m_dim = moments.shape[1]
    h = W_feat.shape[1]
    n = W_out.shape[1]

    u = _sc_gather(user_emb, uid.astype(jnp.int32))

    ae_p = jnp.pad(age_emb, ((0, 8 - age_emb.shape[0]), (0, 0)))
    ge_p = jnp.pad(gender_emb, ((0, 8 - gender_emb.shape[0]), (0, 0)))
    wfa, wmm, wu, tage, tgen, btot = pl.pallas_call(
        _fold_body,
        out_shape=[
            jax.ShapeDtypeStruct((a_dim, n), jnp.bfloat16),
            jax.ShapeDtypeStruct((m_dim, n), jnp.bfloat16),
            jax.ShapeDtypeStruct((h, n), jnp.bfloat16),
            jax.ShapeDtypeStruct((8, n), jnp.float32),
            jax.ShapeDtypeStruct((8, n), jnp.float32),
            jax.ShapeDtypeStruct((1, n), jnp.float32),
        ],
    )(W_feat, b_feat.reshape(1, h), W_mom, b_mom.reshape(1, h),
      ae_p, ge_p, W_out, b_out.reshape(1, n))

    age2 = age.astype(jnp.int32).reshape(b, 1)
    gen2 = gender.astype(jnp.int32).reshape(b, 1)
    bcast = lambda i: (0, 0)
    row = lambda i: (i, 0)
    out = pl.pallas_call(
        _main_body,
        grid=(b // _BB,),
        in_specs=[
            pl.BlockSpec((_BB, a_dim), row),
            pl.BlockSpec((_BB, m_dim), row),
            pl.BlockSpec((_BB, h), row),
            pl.BlockSpec((_BB, 1), row),
            pl.BlockSpec((_BB, 1), row),
            pl.BlockSpec((a_dim, n), bcast),
            pl.BlockSpec((m_dim, n), bcast),
            pl.BlockSpec((h, n), bcast),
            pl.BlockSpec((8, n), bcast),
            pl.BlockSpec((8, n), bcast),
            pl.BlockSpec((1, n), bcast),
        ],
        out_specs=pl.BlockSpec((_BB, n), row),
        out_shape=jax.ShapeDtypeStruct((b, n), jnp.float32),
        compiler_params=pltpu.CompilerParams(
            dimension_semantics=("arbitrary",)),
    )(articles, moments, u, age2, gen2, wfa, wmm, wu, tage, tgen, btot)
    return out


# speedup vs baseline: 1.0368x; 1.0368x over previous
import functools

import jax
import jax.numpy as jnp
from jax import lax
from jax.experimental import pallas as pl
from jax.experimental.pallas import tpu as pltpu
from jax.experimental.pallas import tpu_sc as plsc

_BB = 1024


def _sc_gather(table, idx):
    b, d = idx.shape[0], table.shape[1]
    info = plsc.get_sparse_core_info()
    ncores = info.num_cores
    nwork = ncores * info.num_subcores
    bpw = b // nwork
    mesh = plsc.VectorSubcoreMesh(core_axis_name="c", subcore_axis_name="s")

    @functools.partial(
        pl.kernel,
        mesh=mesh,
        out_type=jax.ShapeDtypeStruct((b, d), table.dtype),
        scratch_types=[
            pltpu.VMEM((bpw,), jnp.int32),
            pltpu.VMEM((bpw, d), table.dtype),
            pltpu.SemaphoreType.DMA,
        ],
    )
    def gather_kernel(table_hbm, idx_hbm, out_hbm, idx_v, rows_v, sem):
        wid = lax.axis_index("s") * ncores + lax.axis_index("c")
        base = wid * bpw
        pltpu.sync_copy(idx_hbm.at[pl.ds(base, bpw)], idx_v)
        pltpu.async_copy(table_hbm.at[idx_v], rows_v, sem).wait()
        pltpu.sync_copy(rows_v, out_hbm.at[pl.ds(base, bpw)])

    return gather_kernel(table, idx)


def _body(art, mom, u, age, gen, wf, bf, wm, bm, ae, ge, wout, bo, out,
          wfa_s, wmm_s, wu_s, tage_s, tgen_s, btot_s):
    h = wf.shape[1]

    @pl.when(pl.program_id(0) == 0)
    def _fold():
        w = wout[...]
        wag = w[h:2 * h, :]
        wg = w[2 * h:3 * h, :]
        wa = w[3 * h:4 * h, :]
        wm2 = w[4 * h:5 * h, :]
        wfa_s[...] = jnp.dot(
            wf[...], wa, preferred_element_type=jnp.float32
        ).astype(jnp.bfloat16)
        wmm_s[...] = jnp.dot(
            wm[...], wm2, preferred_element_type=jnp.float32
        ).astype(jnp.bfloat16)
        wu_s[...] = w[0:h, :].astype(jnp.bfloat16)
        tage_s[...] = jnp.dot(ae[...], wag, preferred_element_type=jnp.float32)
        tgen_s[...] = jnp.dot(ge[...], wg, preferred_element_type=jnp.float32)
        btot_s[...] = (bo[...]
                       + jnp.dot(bf[...], wa, preferred_element_type=jnp.float32)
                       + jnp.dot(bm[...], wm2, preferred_element_type=jnp.float32))

    acc = jnp.dot(art[...].astype(jnp.bfloat16), wfa_s[...],
                  preferred_element_type=jnp.float32)
    acc = acc + jnp.dot(mom[...].astype(jnp.bfloat16), wmm_s[...],
                        preferred_element_type=jnp.float32)
    acc = acc + jnp.dot(u[...].astype(jnp.bfloat16), wu_s[...],
                        preferred_element_type=jnp.float32)
    na, ng = ae.shape[0], ge.shape[0]
    oh_age = (age[...] == lax.broadcasted_iota(jnp.int32, (_BB, na), 1)
              ).astype(jnp.float32)
    acc = acc + jnp.dot(oh_age, tage_s[...], preferred_element_type=jnp.float32)
    oh_gen = (gen[...] == lax.broadcasted_iota(jnp.int32, (_BB, ng), 1)
              ).astype(jnp.float32)
    acc = acc + jnp.dot(oh_gen, tgen_s[...], preferred_element_type=jnp.float32)
    out[...] = acc + btot_s[...]


def kernel(articles, moments, uid, age, gender, W_feat, b_feat, W_mom, b_mom,
           user_emb, age_emb, gender_emb, W_out, b_out):
    b, a_dim = articles.shape
    m_dim = moments.shape[1]
    h = W_feat.shape[1]
    n = W_out.shape[1]

    u = _sc_gather(user_emb, uid.astype(jnp.int32))

    age2 = age.astype(jnp.int32).reshape(b, 1)
    gen2 = gender.astype(jnp.int32).reshape(b, 1)
    na, ng = age_emb.shape[0], gender_emb.shape[0]

    bcast = lambda i: (0, 0)
    row = lambda i: (i, 0)
    out = pl.pallas_call(
        _body,
        grid=(b // _BB,),
        in_specs=[
            pl.BlockSpec((_BB, a_dim), row),
            pl.BlockSpec((_BB, m_dim), row),
            pl.BlockSpec((_BB, h), row),
            pl.BlockSpec((_BB, 1), row),
            pl.BlockSpec((_BB, 1), row),
            pl.BlockSpec((a_dim, h), bcast),
            pl.BlockSpec((1, h), bcast),
            pl.BlockSpec((m_dim, h), bcast),
            pl.BlockSpec((1, h), bcast),
            pl.BlockSpec((na, h), bcast),
            pl.BlockSpec((ng, h), bcast),
            pl.BlockSpec((5 * h, n), bcast),
            pl.BlockSpec((1, n), bcast),
        ],
        out_specs=pl.BlockSpec((_BB, n), row),
        out_shape=jax.ShapeDtypeStruct((b, n), jnp.float32),
        scratch_shapes=[
            pltpu.VMEM((a_dim, n), jnp.bfloat16),
            pltpu.VMEM((m_dim, n), jnp.bfloat16),
            pltpu.VMEM((h, n), jnp.bfloat16),
            pltpu.VMEM((na, n), jnp.float32),
            pltpu.VMEM((ng, n), jnp.float32),
            pltpu.VMEM((1, n), jnp.float32),
        ],
        compiler_params=pltpu.CompilerParams(
            dimension_semantics=("arbitrary",)),
    )(articles, moments, u, age2, gen2, W_feat, b_feat.reshape(1, h),
      W_mom, b_mom.reshape(1, h), age_emb, gender_emb, W_out, b_out.reshape(1, n))
    return out
